# Initial kernel scaffold; baseline (speedup 1.0000x reference)
import jax
import jax.numpy as jnp
from jax.experimental import pallas as pl


def kernel(user_emb, item_emb, edge_val, edge_row, edge_col):
    raise NotImplementedError("write your pallas kernel here")


# speedup vs baseline: 2.7289x; 2.7289x over previous
import functools

import jax
import jax.numpy as jnp
from jax import lax
from jax.experimental import pallas as pl
from jax.experimental.pallas import tpu as pltpu
from jax.experimental.pallas import tpu_sc as plsc

_USER = 25000
_ITEM = 25000
_N = 50000
_E = 800000
_D = 64
_LAYERS = 3

_NC = 2
_NS = 16
_HALF = 25000
_PAD_HALF = 25088
_NFLAT = 2 * _PAD_HALF
_ACC_ROWS = 25216
_TRASH = 25100

_IDXROWS = 6272
_EPAD = _IDXROWS * 128
_ROWS_PER_SUB = _IDXROWS // _NS
_CHUNK_ROWS = 8
_CHUNK = _CHUNK_ROWS * 128
_NCHUNK = _ROWS_PER_SUB // _CHUNK_ROWS
_SUBROWS = 2
_SUB = _SUBROWS * 128

_ZROWS = _ACC_ROWS // _NS
_OROWS = _PAD_HALF // _NS


def _layer_body(ego, col2, row2, val1, out, colv, rowv, valv, rowsv, acc, sem):
    c = lax.axis_index("c")
    s = lax.axis_index("s")
    base_row = c * _HALF

    zero16 = jnp.zeros((16,), jnp.float32)

    def _zbody(i, carry):
        for j in range(_D // 16):
            rowsv[i, pl.ds(j * 16, 16)] = zero16
        return carry

    lax.fori_loop(0, _SUB, _zbody, 0)
    zbase = s * _ZROWS
    for q in range(_ZROWS // _SUB):
        pltpu.sync_copy(
            rowsv.at[pl.ds(0, _SUB)], acc.at[pl.ds(zbase + q * _SUB, _SUB)]
        )
    zrem = _ZROWS % _SUB
    pltpu.sync_copy(
        rowsv.at[pl.ds(0, zrem)],
        acc.at[pl.ds(zbase + _ZROWS - zrem, zrem)],
    )
    plsc.subcore_barrier()

    def _chunk_body(t, carry):
        r0 = s * _ROWS_PER_SUB + t * _CHUNK_ROWS
        e0 = r0 * 128
        pltpu.sync_copy(col2.at[pl.ds(r0, _CHUNK_ROWS)], colv)
        pltpu.sync_copy(row2.at[pl.ds(r0, _CHUNK_ROWS)], rowv)
        pltpu.sync_copy(val1.at[pl.ds(e0, _CHUNK)], valv)

        for j in range(_CHUNK_ROWS):
            for l in range(8):
                cv = colv[j, pl.ds(l * 16, 16)]
                colv[j, pl.ds(l * 16, 16)] = jnp.where(
                    cv >= _HALF, cv + (_PAD_HALF - _HALF), cv
                )
                rv = rowv[j, pl.ds(l * 16, 16)]
                lr = rv - base_row
                ok = (lr >= 0) & (lr < _HALF)
                rowv[j, pl.ds(l * 16, 16)] = jnp.where(ok, lr, _TRASH)

        for sub in range(_CHUNK // _SUB):
            descs = [
                pltpu.async_copy(
                    ego.at[colv.at[sub * _SUBROWS + j]],
                    rowsv.at[pl.ds(j * 128, 128)],
                    sem,
                )
                for j in range(_SUBROWS)
            ]
            for d in descs:
                d.wait()

            def _scale(k, carry2):
                vvec = valv[pl.ds(sub * _SUB + k * 16, 16)]
                base = k * 16
                for l in range(16):
                    v = vvec[l]
                    for j in range(_D // 16):
                        rowsv[base + l, pl.ds(j * 16, 16)] = (
                            rowsv[base + l, pl.ds(j * 16, 16)] * v
                        )
                return carry2

            lax.fori_loop(0, _SUB // 16, _scale, 0)

            for j in range(_SUBROWS):
                pltpu.sync_copy(
                    rowsv.at[pl.ds(j * 128, 128)],
                    acc.at[rowv.at[sub * _SUBROWS + j]],
                    add=True,
                )
        return carry

    lax.fori_loop(0, _NCHUNK, _chunk_body, 0)
    plsc.subcore_barrier()

    obase = s * _OROWS
    pltpu.sync_copy(
        acc.at[pl.ds(obase, _OROWS)],
        out.at[pl.ds(c * _PAD_HALF + obase, _OROWS)],
    )


_layer = functools.partial(
    pl.kernel,
    out_type=jax.ShapeDtypeStruct((_NFLAT, _D), jnp.float32),
    mesh=plsc.VectorSubcoreMesh(
        core_axis_name="c", subcore_axis_name="s", num_cores=_NC,
        num_subcores=_NS,
    ),
    scratch_types=[
        pltpu.VMEM((_CHUNK_ROWS, 128), jnp.int32),
        pltpu.VMEM((_CHUNK_ROWS, 128), jnp.int32),
        pltpu.VMEM((_CHUNK,), jnp.float32),
        pltpu.VMEM((_SUB, _D), jnp.float32),
        pltpu.VMEM_SHARED((_ACC_ROWS, _D), jnp.float32),
        pltpu.SemaphoreType.DMA,
    ],
    compiler_params=pltpu.CompilerParams(use_tc_tiling_on_sc=False),
)(_layer_body)


def _mean_body(a, b, c, d, o):
    o[...] = (a[...] + b[...] + c[...] + d[...]) * 0.25


_mean = pl.pallas_call(
    _mean_body,
    out_shape=jax.ShapeDtypeStruct((_NFLAT, _D), jnp.float32),
    grid=(98,),
    in_specs=[pl.BlockSpec((512, _D), lambda i: (i, 0))] * 4,
    out_specs=pl.BlockSpec((512, _D), lambda i: (i, 0)),
)


def kernel(user_emb, item_emb, edge_val, edge_row, edge_col):
    pad = _EPAD - _E
    col2 = jnp.concatenate(
        [edge_col, jnp.zeros((pad,), jnp.int32)]
    ).reshape(_IDXROWS, 128)
    row2 = jnp.concatenate(
        [edge_row, jnp.zeros((pad,), jnp.int32)]
    ).reshape(_IDXROWS, 128)
    val1 = jnp.concatenate([edge_val, jnp.zeros((pad,), jnp.float32)])

    zpad = jnp.zeros((_PAD_HALF - _HALF, _D), jnp.float32)
    ego = jnp.concatenate([user_emb, zpad, item_emb, zpad], axis=0)

    embs = [ego]
    for _ in range(_LAYERS):
        ego = _layer(ego, col2, row2, val1)
        embs.append(ego)

    mean = _mean(*embs)
    return (mean[:_USER], mean[_PAD_HALF:_PAD_HALF + _ITEM])


# speedup vs baseline: 3.4190x; 1.2529x over previous
import functools

import jax
import jax.numpy as jnp
from jax import lax
from jax.experimental import pallas as pl
from jax.experimental.pallas import tpu as pltpu
from jax.experimental.pallas import tpu_sc as plsc

_USER = 25000
_ITEM = 25000
_N = 50000
_E = 800000
_D = 64
_LAYERS = 3

_NC = 2
_NS = 16
_HALF = 25000
_PAD_HALF = 25088
_NFLAT = 2 * _PAD_HALF
_TRASH = 25056

_SUBW = 96
_NSUB = 8
_NCHUNK = 66
_CHUNK = _SUBW * _NSUB
_RPS = _NSUB * _NCHUNK
_IDXROWS = _NS * _RPS
_EPAD = _IDXROWS * _SUBW

_ZROWS = _PAD_HALF // _NS


def _layer_body(ego, col96, lidx96, val96, out, colv, lidxv, valv, rowsv, acc,
                gidx0, gidx1, gidx2, gidx3, sidx0, sidx1, sidx2, sidx3,
                sem_g, sem_s, sem_t, sem_z):
    c = lax.axis_index("c")
    s = lax.axis_index("s")
    gidx = (gidx0, gidx1, gidx2, gidx3)
    sidx = (sidx0, sidx1, sidx2, sidx3)

    def _idx_copy(dst, src2d, q, j):
        for k in range(_SUBW // 16):
            dst[pl.ds(k * 16, 16)] = src2d[q, j, pl.ds(k * 16, 16)]

    def g_issue(q, j, b):
        _idx_copy(gidx[b], colv, q, j)
        pltpu.async_copy(
            ego.at[gidx[b]], rowsv.at[pl.ds(b * _SUBW, _SUBW)],
            sem_g.at[b],
        )

    def g_wait(b):
        pltpu.make_async_copy(
            ego.at[gidx[b]], rowsv.at[pl.ds(b * _SUBW, _SUBW)],
            sem_g.at[b],
        ).wait()

    def s_issue(b, q, j):
        _idx_copy(sidx[b], lidxv, q, j)
        pltpu.async_copy(
            rowsv.at[pl.ds(b * _SUBW, _SUBW)], acc.at[sidx[b]],
            sem_s.at[b], add=True,
        )

    def s_wait(b):
        pltpu.make_async_copy(
            rowsv.at[pl.ds(b * _SUBW, _SUBW)], acc.at[sidx[b]],
            sem_s.at[b],
        ).wait()

    def stage_issue(t, q):
        r0 = s * _RPS + t * _NSUB
        pltpu.async_copy(col96.at[pl.ds(r0, _NSUB)], colv.at[q], sem_t.at[q])
        pltpu.async_copy(
            lidx96.at[pl.ds(c * _IDXROWS + r0, _NSUB)], lidxv.at[q],
            sem_t.at[q],
        )
        pltpu.async_copy(val96.at[pl.ds(r0, _NSUB)], valv.at[q], sem_t.at[q])

    def stage_wait(q):
        for src, dst in ((col96, colv.at[q]), (lidx96, lidxv.at[q]),
                         (val96, valv.at[q])):
            pltpu.make_async_copy(src.at[pl.ds(0, _NSUB)], dst,
                                  sem_t.at[q]).wait()

    def scale(b, q, sub):
        def body(k, carry):
            vv = valv[q, sub, pl.ds(k * 16, 16)]
            for l in range(16):
                v = vv[l]
                r = b * _SUBW + k * 16 + l
                for jj in range(_D // 16):
                    rowsv[r, pl.ds(jj * 16, 16)] = (
                        rowsv[r, pl.ds(jj * 16, 16)] * v
                    )
            return carry

        lax.fori_loop(0, _SUBW // 16, body, 0)

    zero16 = jnp.zeros((16,), jnp.float32)

    def _zbody(i, carry):
        for j in range(_D // 16):
            rowsv[i, pl.ds(j * 16, 16)] = zero16
        return carry

    lax.fori_loop(0, 4 * _SUBW, _zbody, 0)

    zbase = s * _ZROWS
    nz = _ZROWS // (4 * _SUBW)
    zdescs = []
    for qq in range(nz):
        zdescs.append(pltpu.async_copy(
            rowsv.at[pl.ds(0, 4 * _SUBW)],
            acc.at[pl.ds(zbase + qq * 4 * _SUBW, 4 * _SUBW)], sem_z))
    zrem = _ZROWS - nz * 4 * _SUBW
    zdescs.append(pltpu.async_copy(
        rowsv.at[pl.ds(0, zrem)],
        acc.at[pl.ds(zbase + _ZROWS - zrem, zrem)], sem_z))

    stage_issue(0, 0)
    for d in zdescs:
        d.wait()
    stage_wait(0)
    plsc.subcore_barrier()

    s_issue(2, 0, _NSUB - 2)
    s_issue(3, 0, _NSUB - 1)
    g_issue(0, 0, 0)
    g_issue(0, 1, 1)

    def _pair(m, carry):
        for half in range(2):
            t = 2 * m + half
            q = half
            for sub in range(_NSUB):
                b = sub % 4
                b2 = (sub + 2) % 4
                g_wait(b)
                scale(b, q, sub)
                s_issue(b, q, sub)
                s_wait(b2)
                if sub < _NSUB - 2:
                    g_issue(q, sub + 2, b2)
                elif half == 0:
                    if sub == _NSUB - 2:
                        stage_wait(1)
                    g_issue(1, sub - (_NSUB - 2), b2)
                else:
                    @pl.when(m < (_NCHUNK // 2) - 1)
                    def _():
                        if sub == _NSUB - 2:
                            stage_wait(0)
                        g_issue(0, sub - (_NSUB - 2), b2)
                if sub == 1:
                    if half == 0:
                        stage_issue(t + 1, 1)
                    else:
                        @pl.when(m < (_NCHUNK // 2) - 1)
                        def _():
                            stage_issue(t + 1, 0)
        return carry

    lax.fori_loop(0, _NCHUNK // 2, _pair, 0)

    s_wait(2)
    s_wait(3)
    plsc.subcore_barrier()

    pltpu.sync_copy(
        acc.at[pl.ds(s * _ZROWS, _ZROWS)],
        out.at[pl.ds(c * _PAD_HALF + s * _ZROWS, _ZROWS)],
    )


_layer = functools.partial(
    pl.kernel,
    out_type=jax.ShapeDtypeStruct((_NFLAT, _D), jnp.float32),
    mesh=plsc.VectorSubcoreMesh(
        core_axis_name="c", subcore_axis_name="s", num_cores=_NC,
        num_subcores=_NS,
    ),
    scratch_types=[
        pltpu.VMEM((2, _NSUB, _SUBW), jnp.int32),
        pltpu.VMEM((2, _NSUB, _SUBW), jnp.int32),
        pltpu.VMEM((2, _NSUB, _SUBW), jnp.float32),
        pltpu.VMEM((4 * _SUBW, _D), jnp.float32),
        pltpu.VMEM_SHARED((_PAD_HALF, _D), jnp.float32),
        pltpu.VMEM((_SUBW,), jnp.int32),
        pltpu.VMEM((_SUBW,), jnp.int32),
        pltpu.VMEM((_SUBW,), jnp.int32),
        pltpu.VMEM((_SUBW,), jnp.int32),
        pltpu.VMEM((_SUBW,), jnp.int32),
        pltpu.VMEM((_SUBW,), jnp.int32),
        pltpu.VMEM((_SUBW,), jnp.int32),
        pltpu.VMEM((_SUBW,), jnp.int32),
        pltpu.SemaphoreType.DMA((4,)),
        pltpu.SemaphoreType.DMA((4,)),
        pltpu.SemaphoreType.DMA((2,)),
        pltpu.SemaphoreType.DMA,
    ],
    compiler_params=pltpu.CompilerParams(use_tc_tiling_on_sc=False),
)(_layer_body)


def _mean_body(a, b, c, d, o):
    o[...] = (a[...] + b[...] + c[...] + d[...]) * 0.25


_mean = pl.pallas_call(
    _mean_body,
    out_shape=jax.ShapeDtypeStruct((_NFLAT, _D), jnp.float32),
    grid=(98,),
    in_specs=[pl.BlockSpec((512, _D), lambda i: (i, 0))] * 4,
    out_specs=pl.BlockSpec((512, _D), lambda i: (i, 0)),
)


def kernel(user_emb, item_emb, edge_val, edge_row, edge_col):
    pad = _EPAD - _E
    colp = jnp.concatenate([edge_col, jnp.zeros((pad,), jnp.int32)])
    rowp = jnp.concatenate([edge_row, jnp.zeros((pad,), jnp.int32)])
    valp = jnp.concatenate([edge_val, jnp.zeros((pad,), jnp.float32)])

    col96 = jnp.where(colp >= _HALF, colp + (_PAD_HALF - _HALF), colp)
    col96 = col96.reshape(_IDXROWS, _SUBW)
    lidx0 = jnp.where(rowp < _HALF, rowp, _TRASH)
    lidx1 = jnp.where(rowp >= _HALF, rowp - _HALF, _TRASH)
    lidx96 = jnp.concatenate([lidx0, lidx1]).reshape(2 * _IDXROWS, _SUBW)
    val96 = valp.reshape(_IDXROWS, _SUBW)

    zpad = jnp.zeros((_PAD_HALF - _HALF, _D), jnp.float32)
    ego = jnp.concatenate([user_emb, zpad, item_emb, zpad], axis=0)

    embs = [ego]
    for _ in range(_LAYERS):
        ego = _layer(ego, col96, lidx96, val96)
        embs.append(ego)

    mean = _mean(*embs)
    return (mean[:_USER], mean[_PAD_HALF:_PAD_HALF + _ITEM])
